# baseline (speedup 1.0000x reference)
import functools

import jax
import jax.numpy as jnp
from jax import lax
from jax.experimental import pallas as pl
from jax.experimental.pallas import tpu as pltpu
from jax.experimental.pallas import tpu_sc as plsc

B = 16384
F = 26
D = 32
N = B * F
NC, NS, L = 2, 16, 16
NW = NC * NS
RPW = N // NW
CHUNK_B = 64
CHUNK = CHUNK_B * F
NCHUNK = RPW // CHUNK
GSLICE = 128
NG = CHUNK // GSLICE
HALVES = D // L


def _body(x_hbm, table_hbm, bias_hbm, out_hbm, idx_v, rows_v, bias_v, sem):
    wid = lax.axis_index("s") * NC + lax.axis_index("c")

    pltpu.sync_copy(bias_hbm, bias_v)
    bvecs = [bias_v[f, pl.ds(h * L, L)] for f in range(F) for h in range(HALVES)]

    pltpu.sync_copy(x_hbm.at[pl.ds(wid * (RPW // GSLICE), RPW // GSLICE)], idx_v)

    for c in range(NCHUNK):
        base = wid * RPW + c * CHUNK
        descs = [
            pltpu.async_copy(table_hbm.at[idx_v.at[c * NG + j]],
                             rows_v.at[pl.ds(j * GSLICE, GSLICE)], sem)
            for j in range(NG)
        ]
        for d in descs:
            d.wait()

        def add_bias(r, carry):
            base_row = r * F
            for f in range(F):
                for h in range(HALVES):
                    sl = pl.ds(h * L, L)
                    rows_v[base_row + f, sl] = (
                        rows_v[base_row + f, sl] + bvecs[f * HALVES + h])
            return carry

        lax.fori_loop(0, CHUNK_B, add_bias, 0)

        pltpu.sync_copy(rows_v, out_hbm.at[pl.ds(base, CHUNK)])


@jax.jit
def _embed(x_flat2, table, bias):
    mesh = plsc.VectorSubcoreMesh(core_axis_name="c", subcore_axis_name="s")
    f = pl.kernel(
        _body,
        out_type=jax.ShapeDtypeStruct((N, D), jnp.float32),
        mesh=mesh,
        compiler_params=pltpu.CompilerParams(use_tc_tiling_on_sc=False),
        scratch_types=[
            pltpu.VMEM((RPW // GSLICE, GSLICE), jnp.int32),
            pltpu.VMEM((CHUNK, D), jnp.float32),
            pltpu.VMEM((F, D), jnp.float32),
            pltpu.SemaphoreType.DMA,
        ],
    )
    return f(x_flat2, table, bias)


def kernel(X, table, bias):
    x_flat2 = X.reshape(N // GSLICE, GSLICE)
    out = _embed(x_flat2, table, bias)
    return out.reshape(B, F, D)
